# Initial kernel scaffold; baseline (speedup 1.0000x reference)
import functools

import jax
import jax.numpy as jnp
from jax import lax
from jax.experimental import pallas as pl
from jax.experimental.pallas import tpu as pltpu
from jax.experimental.pallas import tpu_sc as plsc

N = 10000
E = 320000
H = 128
L = 3
G = 128
C = 0.001

NC = 2
NS = 16
NW = NC * NS
EPW = E // NW
GB = 80
NJ = EPW // GB
NBUF = 5

_f32 = jnp.float32


def _psi(p):
    return jnp.sign(p) * jnp.log(jnp.abs(p) + 1.0)


_sc_mesh = plsc.VectorSubcoreMesh(core_axis_name="c", subcore_axis_name="s")


@functools.partial(
    pl.kernel,
    out_type=jax.ShapeDtypeStruct((NW, 2 * NJ, GB, 4), _f32),
    mesh=_sc_mesh,
    scratch_types=(
        [pltpu.VMEM((2 * NJ, GB), jnp.int32)]
        + [pltpu.VMEM((GB, 4), _f32) for _ in range(NBUF)]
        + [pltpu.SemaphoreType.DMA for _ in range(NBUF)]
    ),
)
def _sc_gather(x_hbm, idx_hbm, out_hbm, idx_v, *bufs_and_sems):
    rows = bufs_and_sems[:NBUF]
    sems = bufs_and_sems[NBUF:]
    wid = lax.axis_index("s") * NC + lax.axis_index("c")
    pltpu.sync_copy(idx_hbm.at[wid], idx_v)
    for b in range(NBUF):
        pltpu.async_copy(x_hbm.at[idx_v.at[b]], rows[b], sems[b])

    def group(g, carry):
        for b in range(NBUF):
            jf = g * NBUF + b
            pltpu.make_async_copy(x_hbm.at[idx_v.at[jf]], rows[b], sems[b]).wait()
            pltpu.sync_copy(rows[b], out_hbm.at[wid, jf])
            jn = jf + NBUF

            @pl.when(jn < 2 * NJ)
            def _():
                pltpu.async_copy(x_hbm.at[idx_v.at[jn]], rows[b], sems[b])

        return carry

    lax.fori_loop(0, (2 * NJ) // NBUF, group, 0)


@functools.partial(
    pl.kernel,
    out_type=jax.ShapeDtypeStruct((NC, N, 4), _f32),
    mesh=_sc_mesh,
    scratch_types=(
        pltpu.VMEM((NJ, GB, 4), _f32),
        pltpu.VMEM((NJ, GB), jnp.int32),
        pltpu.VMEM_SHARED((N, 4), _f32),
    ),
)
def _sc_scatter(x_hbm, z_hbm, msg_hbm, dst_hbm, out_hbm, msg_v, idx_v, acc):
    cid = lax.axis_index("c")
    sid = lax.axis_index("s")
    wid = sid * NC + cid

    @pl.when(sid == 0)
    def _():
        @pl.when(cid == 0)
        def _():
            pltpu.sync_copy(x_hbm, acc)

        @pl.when(cid != 0)
        def _():
            pltpu.sync_copy(z_hbm, acc)

    plsc.subcore_barrier()
    pltpu.sync_copy(msg_hbm.at[wid], msg_v)
    pltpu.sync_copy(dst_hbm.at[wid], idx_v)

    def body(j, carry):
        pltpu.sync_copy(msg_v.at[j], acc.at[idx_v.at[j]], add=True)
        return carry

    lax.fori_loop(0, NJ, body, 0)
    plsc.subcore_barrier()

    @pl.when(sid == 0)
    def _():
        pltpu.sync_copy(acc, out_hbm.at[cid])


BS = 8000
BM = 4000
_METRIC = (1.0, -1.0, -1.0, -1.0)


def _stats_body(xs_ref, xd_ref, min_ref, md_ref, st_ref, acc):
    i = pl.program_id(0)
    xs = xs_ref[...]
    xd = xd_ref[...]
    metric = jnp.array([_METRIC], _f32)
    d = xs - xd
    m1 = _psi(jnp.sum(d * d * metric, axis=1, keepdims=True))
    m2 = _psi(jnp.sum(xs * xd * metric, axis=1, keepdims=True))
    min_ref[:, 0:1] = m1
    min_ref[:, 1:2] = m2
    md_ref[...] = xd - xs

    @pl.when(i == 0)
    def _():
        acc[...] = jnp.zeros_like(acc)

    acc[:, 0:1] += jnp.sum(m1, axis=0, keepdims=True)
    acc[:, 1:2] += jnp.sum(m2, axis=0, keepdims=True)
    acc[:, 2:3] += jnp.sum(m1 * m1, axis=0, keepdims=True)
    acc[:, 3:4] += jnp.sum(m1 * m2, axis=0, keepdims=True)
    acc[:, 4:5] += jnp.sum(m2 * m2, axis=0, keepdims=True)

    @pl.when(i == pl.num_programs(0) - 1)
    def _():
        st_ref[...] = acc[...]


def _tc_stats(xs, xd):
    return pl.pallas_call(
        _stats_body,
        grid=(E // BS,),
        in_specs=[
            pl.BlockSpec((BS, 4), lambda i: (i, 0)),
            pl.BlockSpec((BS, 4), lambda i: (i, 0)),
        ],
        out_specs=[
            pl.BlockSpec((BS, 2), lambda i: (i, 0)),
            pl.BlockSpec((BS, 4), lambda i: (i, 0)),
            pl.BlockSpec((1, 8), lambda i: (0, 0)),
        ],
        out_shape=[
            jax.ShapeDtypeStruct((E, 2), _f32),
            jax.ShapeDtypeStruct((E, 4), _f32),
            jax.ShapeDtypeStruct((1, 8), _f32),
        ],
        scratch_shapes=[pltpu.VMEM((1, 8), _f32)],
    )(xs, xd)


def _mlp_body(min_ref, md_ref, w1_ref, b1_ref, w2_ref, b2_ref, wm_ref, bm_ref,
              wx1_ref, bx1_ref, wx2_ref, out_ref):
    m = min_ref[...]
    w1 = w1_ref[...]
    h = m[:, 0:1] * w1[0:1, :] + m[:, 1:2] * w1[1:2, :] + b1_ref[...]
    h = jnp.maximum(h, 0.0)
    mij = jnp.dot(h, w2_ref[...], preferred_element_type=_f32) + b2_ref[...]
    mij = jnp.maximum(mij, 0.0)
    gate = jax.nn.sigmoid(
        jnp.sum(mij * wm_ref[...], axis=1, keepdims=True) + bm_ref[0, 0])
    mij = mij * gate
    t = jnp.dot(mij, wx1_ref[...], preferred_element_type=_f32) + bx1_ref[...]
    t = jnp.maximum(t, 0.0)
    scale = jnp.sum(t * wx2_ref[...], axis=1, keepdims=True)
    out_ref[...] = (C * scale) * md_ref[...]


def _tc_mlp(m_in, md, w1p, b1p, w2, b2, wm, bm, wx1, bx1, wx2):
    def full(r, c):
        return pl.BlockSpec((r, c), lambda i: (0, 0))
    return pl.pallas_call(
        _mlp_body,
        grid=(E // BM,),
        in_specs=[
            pl.BlockSpec((BM, 2), lambda i: (i, 0)),
            pl.BlockSpec((BM, 4), lambda i: (i, 0)),
            full(2, H), full(1, H), full(H, H), full(1, H),
            full(1, H), full(1, 1), full(H, H), full(1, H), full(1, H),
        ],
        out_specs=pl.BlockSpec((BM, 4), lambda i: (i, 0)),
        out_shape=jax.ShapeDtypeStruct((E, 4), _f32),
    )(m_in, md, w1p, b1p, w2, b2, wm, bm, wx1, bx1, wx2)


def _pool_body(x_ref, b_ref, wd1_ref, bd1_ref, wd2_ref, bd2_ref, out_ref):
    xv = x_ref[...]
    bv = b_ref[...]
    oh = (bv == lax.broadcasted_iota(jnp.int32, (1, G), 1)).astype(_f32)
    sums = lax.dot_general(oh, xv, (((0,), (0,)), ((), ())),
                           preferred_element_type=_f32)
    cnt = lax.dot_general(oh, jnp.ones((N, 1), _f32), (((0,), (0,)), ((), ())),
                          preferred_element_type=_f32)
    avg = sums / jnp.maximum(cnt, 1.0)
    h = jnp.dot(avg, wd1_ref[...], preferred_element_type=_f32) + bd1_ref[...]
    h = jax.nn.gelu(h, approximate=False)
    out_ref[...] = jnp.sum(h * wd2_ref[...], axis=1, keepdims=True) + bd2_ref[0, 0]


def _tc_pool(x, batch_col, wd1, bd1, wd2, bd2):
    return pl.pallas_call(
        _pool_body,
        out_shape=jax.ShapeDtypeStruct((G, 1), _f32),
    )(x, batch_col, wd1, bd1, wd2, bd2)


def _fold_bn(st, w_e1, g, b):
    s = st[0]
    mbar1, mbar2 = s[0] / E, s[1] / E
    s11, s12, s22 = s[2] / E, s[3] / E, s[4] / E
    w0, w1 = w_e1[0], w_e1[1]
    mu = mbar1 * w0 + mbar2 * w1
    ex2 = w0 * w0 * s11 + 2.0 * w0 * w1 * s12 + w1 * w1 * s22
    var = ex2 - mu * mu
    rstd = g * jax.lax.rsqrt(var + 1e-5)
    w1p = w_e1 * rstd[None, :]
    b1p = b - mu * rstd
    return w1p, b1p[None, :]


def kernel(x, edge_index, batch, W_e1, bn_g, bn_b, W_e2, b_e2, W_x1, b_x1,
           W_x2, W_m, b_m, Wd1, bd1, Wd2, bd2):
    src = edge_index[0].astype(jnp.int32)
    dst = edge_index[1].astype(jnp.int32)
    idx_all = jnp.concatenate(
        [src.reshape(NW, NJ, GB), dst.reshape(NW, NJ, GB)], axis=1)
    dst_w = dst.reshape(NW, NJ, GB)
    zeros_n4 = jnp.zeros((N, 4), _f32)
    batch_col = batch.astype(jnp.int32).reshape(N, 1)
    x = x.astype(_f32)

    for l in range(L):
        gath = _sc_gather(x, idx_all)
        xs = gath[:, :NJ].reshape(E, 4)
        xd = gath[:, NJ:].reshape(E, 4)
        m_in, md, st = _tc_stats(xs, xd)
        w1p, b1p = _fold_bn(st, W_e1[l], bn_g[l], bn_b[l])
        msg = _tc_mlp(m_in, md, w1p, b1p, W_e2[l], b_e2[l][None, :],
                      W_m[l].reshape(1, H), b_m[l].reshape(1, 1),
                      W_x1[l], b_x1[l][None, :], W_x2[l].reshape(1, H))
        parts = _sc_scatter(x, zeros_n4, msg.reshape(NW, NJ, GB, 4), dst_w)
        x = parts[0] + parts[1]

    return _tc_pool(x, batch_col, Wd1, bd1[None, :], Wd2.reshape(1, H),
                    bd2.reshape(1, 1))


# speedup vs baseline: 2.5416x; 2.5416x over previous
import functools

import jax
import jax.numpy as jnp
from jax import lax
from jax.experimental import pallas as pl
from jax.experimental.pallas import tpu as pltpu
from jax.experimental.pallas import tpu_sc as plsc

N = 10000
E = 320000
H = 128
L = 3
G = 128
C = 0.001

NC = 2
NS = 16
NW = NC * NS
EPW = E // NW
GB = 80
NJ = EPW // GB
NBUF = 5
DP = 8

_f32 = jnp.float32


def _psi(p):
    return jnp.sign(p) * jnp.log(jnp.abs(p) + 1.0)


def _fill_idx(dst1d, src3d, j):
    for k in range(GB // 16):
        dst1d[pl.ds(k * 16, 16)] = src3d[j, k]


def _sc_gather_body(x_hbm, idx_hbm, out_hbm, idx_v, *bufs_and_sems):
    idxb = bufs_and_sems[:NBUF]
    rows = bufs_and_sems[NBUF:2 * NBUF]
    sems = bufs_and_sems[2 * NBUF:]
    wid = lax.axis_index("s") * NC + lax.axis_index("c")
    pltpu.sync_copy(idx_hbm.at[wid], idx_v)
    for b in range(NBUF):
        _fill_idx(idxb[b], idx_v, b)
        pltpu.async_copy(x_hbm.at[idxb[b]], rows[b], sems[b])

    def group(g, carry):
        for b in range(NBUF):
            jf = g * NBUF + b
            pltpu.make_async_copy(x_hbm.at[idxb[b]], rows[b], sems[b]).wait()
            pltpu.sync_copy(rows[b], out_hbm.at[wid, jf])
            jn = jf + NBUF

            @pl.when(jn < 2 * NJ)
            def _():
                _fill_idx(idxb[b], idx_v, jn)
                pltpu.async_copy(x_hbm.at[idxb[b]], rows[b], sems[b])

        return carry

    lax.fori_loop(0, (2 * NJ) // NBUF, group, 0)


def _sc_scatter_body(x_hbm, z_hbm, msg_hbm, dst_hbm, out_hbm, msg_v, idx_v,
                     idxb, acc):
    cid = lax.axis_index("c")
    sid = lax.axis_index("s")
    wid = sid * NC + cid

    @pl.when(sid == 0)
    def _():
        @pl.when(cid == 0)
        def _():
            pltpu.sync_copy(x_hbm, acc)

        @pl.when(cid != 0)
        def _():
            pltpu.sync_copy(z_hbm, acc)

    plsc.subcore_barrier()
    pltpu.sync_copy(msg_hbm.at[wid], msg_v)
    pltpu.sync_copy(dst_hbm.at[wid], idx_v)

    def body(j, carry):
        _fill_idx(idxb, idx_v, j)
        pltpu.sync_copy(msg_v.at[j], acc.at[idxb], add=True)
        return carry

    lax.fori_loop(0, NJ, body, 0)
    plsc.subcore_barrier()

    @pl.when(sid == 0)
    def _():
        pltpu.sync_copy(acc, out_hbm.at[cid])


@functools.cache
def _build_sc_kernels():
    mesh = plsc.VectorSubcoreMesh(
        core_axis_name="c", subcore_axis_name="s",
        num_cores=NC, num_subcores=NS)
    params = pltpu.CompilerParams(use_tc_tiling_on_sc=False)
    gather = pl.kernel(
        _sc_gather_body,
        out_type=jax.ShapeDtypeStruct((NW, 2 * NJ, GB, DP), _f32),
        mesh=mesh,
        compiler_params=params,
        scratch_types=(
            [pltpu.VMEM((2 * NJ, GB // 16, 16), jnp.int32)]
            + [pltpu.VMEM((GB,), jnp.int32) for _ in range(NBUF)]
            + [pltpu.VMEM((GB, DP), _f32) for _ in range(NBUF)]
            + [pltpu.SemaphoreType.DMA for _ in range(NBUF)]
        ),
    )
    scatter = pl.kernel(
        _sc_scatter_body,
        out_type=jax.ShapeDtypeStruct((NC, N, DP), _f32),
        mesh=mesh,
        compiler_params=params,
        scratch_types=(
            pltpu.VMEM((NJ, GB, DP), _f32),
            pltpu.VMEM((NJ, GB // 16, 16), jnp.int32),
            pltpu.VMEM((GB,), jnp.int32),
            pltpu.VMEM_SHARED((N, DP), _f32),
        ),
    )
    return gather, scatter


BS = 8000
BM = 4000
_METRIC = (1.0, -1.0, -1.0, -1.0)


def _stats_body(xs_ref, xd_ref, min_ref, md_ref, st_ref, acc):
    i = pl.program_id(0)
    xs = xs_ref[...][:, 0:4]
    xd = xd_ref[...][:, 0:4]
    metric = jnp.where(
        lax.broadcasted_iota(jnp.int32, (1, 4), 1) == 0, 1.0, -1.0)
    d = xs - xd
    m1 = _psi(jnp.sum(d * d * metric, axis=1, keepdims=True))
    m2 = _psi(jnp.sum(xs * xd * metric, axis=1, keepdims=True))
    min_ref[:, 0:1] = m1
    min_ref[:, 1:2] = m2
    md_ref[...] = xd - xs

    @pl.when(i == 0)
    def _():
        acc[...] = jnp.zeros_like(acc)

    acc[:, 0:1] += jnp.sum(m1, axis=0, keepdims=True)
    acc[:, 1:2] += jnp.sum(m2, axis=0, keepdims=True)
    acc[:, 2:3] += jnp.sum(m1 * m1, axis=0, keepdims=True)
    acc[:, 3:4] += jnp.sum(m1 * m2, axis=0, keepdims=True)
    acc[:, 4:5] += jnp.sum(m2 * m2, axis=0, keepdims=True)

    @pl.when(i == pl.num_programs(0) - 1)
    def _():
        st_ref[...] = acc[...]


def _tc_stats(xs, xd):
    return pl.pallas_call(
        _stats_body,
        grid=(E // BS,),
        in_specs=[
            pl.BlockSpec((BS, DP), lambda i: (i, 0)),
            pl.BlockSpec((BS, DP), lambda i: (i, 0)),
        ],
        out_specs=[
            pl.BlockSpec((BS, 2), lambda i: (i, 0)),
            pl.BlockSpec((BS, 4), lambda i: (i, 0)),
            pl.BlockSpec((1, 8), lambda i: (0, 0)),
        ],
        out_shape=[
            jax.ShapeDtypeStruct((E, 2), _f32),
            jax.ShapeDtypeStruct((E, 4), _f32),
            jax.ShapeDtypeStruct((1, 8), _f32),
        ],
        scratch_shapes=[pltpu.VMEM((1, 8), _f32)],
    )(xs, xd)


def _mlp_body(min_ref, md_ref, w1_ref, b1_ref, w2_ref, b2_ref, wm_ref, bm_ref,
              wx1_ref, bx1_ref, wx2_ref, out_ref):
    m = min_ref[...]
    w1 = w1_ref[...]
    h = m[:, 0:1] * w1[0:1, :] + m[:, 1:2] * w1[1:2, :] + b1_ref[...]
    h = jnp.maximum(h, 0.0)
    mij = jnp.dot(h, w2_ref[...], preferred_element_type=_f32) + b2_ref[...]
    mij = jnp.maximum(mij, 0.0)
    gate = jax.nn.sigmoid(
        jnp.sum(mij * wm_ref[...], axis=1, keepdims=True) + bm_ref[0, 0])
    mij = mij * gate
    t = jnp.dot(mij, wx1_ref[...], preferred_element_type=_f32) + bx1_ref[...]
    t = jnp.maximum(t, 0.0)
    scale = jnp.sum(t * wx2_ref[...], axis=1, keepdims=True)
    out_ref[:, 0:4] = (C * scale) * md_ref[...]
    out_ref[:, 4:DP] = jnp.zeros((out_ref.shape[0], DP - 4), _f32)


def _tc_mlp(m_in, md, w1p, b1p, w2, b2, wm, bm, wx1, bx1, wx2):
    def full(r, c):
        return pl.BlockSpec((r, c), lambda i: (0, 0))
    return pl.pallas_call(
        _mlp_body,
        grid=(E // BM,),
        in_specs=[
            pl.BlockSpec((BM, 2), lambda i: (i, 0)),
            pl.BlockSpec((BM, 4), lambda i: (i, 0)),
            full(2, H), full(1, H), full(H, H), full(1, H),
            full(1, H), full(1, 1), full(H, H), full(1, H), full(1, H),
        ],
        out_specs=pl.BlockSpec((BM, DP), lambda i: (i, 0)),
        out_shape=jax.ShapeDtypeStruct((E, DP), _f32),
    )(m_in, md, w1p, b1p, w2, b2, wm, bm, wx1, bx1, wx2)


def _erf(x):
    s = jnp.sign(x)
    a = jnp.abs(x)
    t = 1.0 / (1.0 + 0.3275911 * a)
    poly = t * (0.254829592 + t * (-0.284496736 + t * (1.421413741
           + t * (-1.453152027 + t * 1.061405429))))
    return s * (1.0 - poly * jnp.exp(-a * a))


def _gelu_exact(x):
    return 0.5 * x * (1.0 + _erf(x * 0.7071067811865476))


def _pool_body(x_ref, b_ref, wd1_ref, bd1_ref, wd2_ref, bd2_ref, out_ref):
    xv = x_ref[...]
    bv = b_ref[...]
    oh = (bv == lax.broadcasted_iota(jnp.int32, (1, G), 1)).astype(_f32)
    sums = lax.dot_general(oh, xv, (((0,), (0,)), ((), ())),
                           preferred_element_type=_f32)[:, 0:4]
    cnt = lax.dot_general(oh, jnp.ones((N, 1), _f32), (((0,), (0,)), ((), ())),
                          preferred_element_type=_f32)
    avg = sums / jnp.maximum(cnt, 1.0)
    h = jnp.dot(avg, wd1_ref[...], preferred_element_type=_f32) + bd1_ref[...]
    h = _gelu_exact(h)
    out_ref[...] = jnp.sum(h * wd2_ref[...], axis=1, keepdims=True) + bd2_ref[0, 0]


def _tc_pool(x, batch_col, wd1, bd1, wd2, bd2):
    return pl.pallas_call(
        _pool_body,
        out_shape=jax.ShapeDtypeStruct((G, 1), _f32),
    )(x, batch_col, wd1, bd1, wd2, bd2)


def _fold_bn(st, w_e1, g, b):
    s = st[0]
    mbar1, mbar2 = s[0] / E, s[1] / E
    s11, s12, s22 = s[2] / E, s[3] / E, s[4] / E
    w0, w1 = w_e1[0], w_e1[1]
    mu = mbar1 * w0 + mbar2 * w1
    ex2 = w0 * w0 * s11 + 2.0 * w0 * w1 * s12 + w1 * w1 * s22
    var = ex2 - mu * mu
    rstd = g * jax.lax.rsqrt(var + 1e-5)
    w1p = w_e1 * rstd[None, :]
    b1p = b - mu * rstd
    return w1p, b1p[None, :]


def kernel(x, edge_index, batch, W_e1, bn_g, bn_b, W_e2, b_e2, W_x1, b_x1,
           W_x2, W_m, b_m, Wd1, bd1, Wd2, bd2):
    src = edge_index[0].astype(jnp.int32)
    dst = edge_index[1].astype(jnp.int32)
    idx_all = jnp.concatenate(
        [src.reshape(NW, NJ, GB), dst.reshape(NW, NJ, GB)],
        axis=1).reshape(NW, 2 * NJ, GB // 16, 16)
    dst_w = dst.reshape(NW, NJ, GB // 16, 16)
    zeros_n = jnp.zeros((N, DP), _f32)
    batch_col = batch.astype(jnp.int32).reshape(N, 1)
    x = jnp.concatenate(
        [x.astype(_f32), jnp.zeros((N, DP - 4), _f32)], axis=1)
    _sc_gather, _sc_scatter = _build_sc_kernels()

    for l in range(L):
        gath = _sc_gather(x, idx_all)
        xs = gath[:, :NJ].reshape(E, DP)
        xd = gath[:, NJ:].reshape(E, DP)
        m_in, md, st = _tc_stats(xs, xd)
        w1p, b1p = _fold_bn(st, W_e1[l], bn_g[l], bn_b[l])
        msg = _tc_mlp(m_in, md, w1p, b1p, W_e2[l], b_e2[l][None, :],
                      W_m[l].reshape(1, H), b_m[l].reshape(1, 1),
                      W_x1[l], b_x1[l][None, :], W_x2[l].reshape(1, H))
        parts = _sc_scatter(x, zeros_n, msg.reshape(NW, NJ, GB, DP), dst_w)
        x = parts[0] + parts[1]

    return _tc_pool(x, batch_col, Wd1, bd1[None, :], Wd2.reshape(1, H),
                    bd2.reshape(1, 1))


# speedup vs baseline: 3.0503x; 1.2002x over previous
import functools

import jax
import jax.numpy as jnp
from jax import lax
from jax.experimental import pallas as pl
from jax.experimental.pallas import tpu as pltpu
from jax.experimental.pallas import tpu_sc as plsc

N = 10000
E = 320000
H = 128
L = 3
G = 128
C = 0.001

NC = 2
NS = 16
NW = NC * NS
EPW = E // NW
GB = 80
NJ = EPW // GB
NBUF = 5
DP = 8

_f32 = jnp.float32


def _psi(p):
    return jnp.sign(p) * jnp.log(jnp.abs(p) + 1.0)


def _fill_idx(dst1d, src3d, j):
    for k in range(GB // 16):
        dst1d[pl.ds(k * 16, 16)] = src3d[j, k]


def _sc_gather_body(x_hbm, idx_hbm, out_hbm, idx_v, *bufs_and_sems):
    idxb = bufs_and_sems[:NBUF]
    rows = bufs_and_sems[NBUF:2 * NBUF]
    sems = bufs_and_sems[2 * NBUF:]
    wid = lax.axis_index("s") * NC + lax.axis_index("c")
    pltpu.sync_copy(idx_hbm.at[wid], idx_v)
    for b in range(NBUF):
        _fill_idx(idxb[b], idx_v, b)
        pltpu.async_copy(x_hbm.at[idxb[b]], rows[b], sems[b])

    def group(g, carry):
        for b in range(NBUF):
            jf = g * NBUF + b
            pltpu.make_async_copy(x_hbm.at[idxb[b]], rows[b], sems[b]).wait()
            pltpu.sync_copy(rows[b], out_hbm.at[wid, jf])
            jn = jf + NBUF

            @pl.when(jn < 2 * NJ)
            def _():
                _fill_idx(idxb[b], idx_v, jn)
                pltpu.async_copy(x_hbm.at[idxb[b]], rows[b], sems[b])

        return carry

    lax.fori_loop(0, (2 * NJ) // NBUF, group, 0)


def _sc_scatter_body(x_hbm, z_hbm, msg_hbm, dst_hbm, out_hbm, msg_v, idx_v,
                     idxb, acc):
    cid = lax.axis_index("c")
    sid = lax.axis_index("s")
    wid = sid * NC + cid

    @pl.when(sid == 0)
    def _():
        @pl.when(cid == 0)
        def _():
            pltpu.sync_copy(x_hbm, acc)

        @pl.when(cid != 0)
        def _():
            pltpu.sync_copy(z_hbm, acc)

    plsc.subcore_barrier()
    pltpu.sync_copy(msg_hbm.at[wid], msg_v)
    pltpu.sync_copy(dst_hbm.at[wid], idx_v)

    def body(j, carry):
        _fill_idx(idxb, idx_v, j)
        pltpu.sync_copy(msg_v.at[j], acc.at[idxb], add=True)
        return carry

    lax.fori_loop(0, NJ, body, 0)
    plsc.subcore_barrier()

    @pl.when(sid == 0)
    def _():
        pltpu.sync_copy(acc, out_hbm.at[cid])


@functools.cache
def _build_sc_kernels():
    mesh = plsc.VectorSubcoreMesh(
        core_axis_name="c", subcore_axis_name="s",
        num_cores=NC, num_subcores=NS)
    params = pltpu.CompilerParams(use_tc_tiling_on_sc=False)
    gather = pl.kernel(
        _sc_gather_body,
        out_type=jax.ShapeDtypeStruct((NW, 2 * NJ, GB, DP), _f32),
        mesh=mesh,
        compiler_params=params,
        scratch_types=(
            [pltpu.VMEM((2 * NJ, GB // 16, 16), jnp.int32)]
            + [pltpu.VMEM((GB,), jnp.int32) for _ in range(NBUF)]
            + [pltpu.VMEM((GB, DP), _f32) for _ in range(NBUF)]
            + [pltpu.SemaphoreType.DMA for _ in range(NBUF)]
        ),
    )
    scatter = pl.kernel(
        _sc_scatter_body,
        out_type=jax.ShapeDtypeStruct((NC, N, DP), _f32),
        mesh=mesh,
        compiler_params=params,
        scratch_types=(
            pltpu.VMEM((NJ, GB, DP), _f32),
            pltpu.VMEM((NJ, GB // 16, 16), jnp.int32),
            pltpu.VMEM((GB,), jnp.int32),
            pltpu.VMEM_SHARED((N, DP), _f32),
        ),
    )
    return gather, scatter


BS = 8000
BM = 4000
_METRIC = (1.0, -1.0, -1.0, -1.0)


def _stats_body(xs_ref, xd_ref, min_ref, md_ref, st_ref, acc):
    i = pl.program_id(0)
    xs = xs_ref[0, 0][:, 0:4]
    xd = xd_ref[0, 0][:, 0:4]
    metric = jnp.where(
        lax.broadcasted_iota(jnp.int32, (1, 4), 1) == 0, 1.0, -1.0)
    d = xs - xd
    m1 = _psi(jnp.sum(d * d * metric, axis=1, keepdims=True))
    m2 = _psi(jnp.sum(xs * xd * metric, axis=1, keepdims=True))
    min_ref[0, :, 0:1] = m1
    min_ref[0, :, 1:2] = m2
    md_ref[0] = xd - xs

    @pl.when(i == 0)
    def _():
        acc[...] = jnp.zeros_like(acc)

    acc[:, 0:1] += jnp.sum(m1, axis=0, keepdims=True)
    acc[:, 1:2] += jnp.sum(m2, axis=0, keepdims=True)
    acc[:, 2:3] += jnp.sum(m1 * m1, axis=0, keepdims=True)
    acc[:, 3:4] += jnp.sum(m1 * m2, axis=0, keepdims=True)
    acc[:, 4:5] += jnp.sum(m2 * m2, axis=0, keepdims=True)

    @pl.when(i == pl.num_programs(0) - 1)
    def _():
        st_ref[...] = acc[...]


def _tc_stats(gv):
    return pl.pallas_call(
        _stats_body,
        grid=(NW,),
        in_specs=[
            pl.BlockSpec((1, 1, EPW, DP), lambda w: (w, 0, 0, 0)),
            pl.BlockSpec((1, 1, EPW, DP), lambda w: (w, 1, 0, 0)),
        ],
        out_specs=[
            pl.BlockSpec((1, EPW, 2), lambda w: (w, 0, 0)),
            pl.BlockSpec((1, EPW, 4), lambda w: (w, 0, 0)),
            pl.BlockSpec((1, 8), lambda w: (0, 0)),
        ],
        out_shape=[
            jax.ShapeDtypeStruct((NW, EPW, 2), _f32),
            jax.ShapeDtypeStruct((NW, EPW, 4), _f32),
            jax.ShapeDtypeStruct((1, 8), _f32),
        ],
        scratch_shapes=[pltpu.VMEM((1, 8), _f32)],
    )(gv, gv)


def _mlp_body(min_ref, md_ref, w1_ref, b1_ref, w2_ref, b2_ref, wm_ref, bm_ref,
              wx1_ref, bx1_ref, wx2_ref, out_ref):
    m = min_ref[0]
    w1 = w1_ref[...]
    h = m[:, 0:1] * w1[0:1, :] + m[:, 1:2] * w1[1:2, :] + b1_ref[...]
    h = jnp.maximum(h, 0.0)
    mij = jnp.dot(h, w2_ref[...], preferred_element_type=_f32) + b2_ref[...]
    mij = jnp.maximum(mij, 0.0)
    gate = jax.nn.sigmoid(
        jnp.dot(mij, wm_ref[...], preferred_element_type=_f32) + bm_ref[0, 0])
    mij = mij * gate
    t = jnp.dot(mij, wx1_ref[...], preferred_element_type=_f32) + bx1_ref[...]
    t = jnp.maximum(t, 0.0)
    scale = jnp.dot(t, wx2_ref[...], preferred_element_type=_f32)
    out_ref[0, :, 0:4] = (C * scale) * md_ref[0]
    out_ref[0, :, 4:DP] = jnp.zeros((out_ref.shape[1], DP - 4), _f32)


SP = 2
BM2 = EPW // SP


def _tc_mlp(m_in, md, w1p, b1p, w2, b2, wm, bm, wx1, bx1, wx2):
    def full(r, c):
        return pl.BlockSpec((r, c), lambda w, s: (0, 0))
    return pl.pallas_call(
        _mlp_body,
        grid=(NW, SP),
        in_specs=[
            pl.BlockSpec((1, BM2, 2), lambda w, s: (w, s, 0)),
            pl.BlockSpec((1, BM2, 4), lambda w, s: (w, s, 0)),
            full(2, H), full(1, H), full(H, H), full(1, H),
            full(H, 1), full(1, 1), full(H, H), full(1, H), full(H, 1),
        ],
        out_specs=pl.BlockSpec((1, BM2, DP), lambda w, s: (w, s, 0)),
        out_shape=jax.ShapeDtypeStruct((NW, EPW, DP), _f32),
    )(m_in, md, w1p, b1p, w2, b2, wm, bm, wx1, bx1, wx2)


def _erf(x):
    s = jnp.sign(x)
    a = jnp.abs(x)
    t = 1.0 / (1.0 + 0.3275911 * a)
    poly = t * (0.254829592 + t * (-0.284496736 + t * (1.421413741
           + t * (-1.453152027 + t * 1.061405429))))
    return s * (1.0 - poly * jnp.exp(-a * a))


def _gelu_exact(x):
    return 0.5 * x * (1.0 + _erf(x * 0.7071067811865476))


def _pool_body(x_ref, b_ref, wd1_ref, bd1_ref, wd2_ref, bd2_ref, out_ref):
    xv = x_ref[...]
    bv = b_ref[...]
    oh = (bv == lax.broadcasted_iota(jnp.int32, (1, G), 1)).astype(_f32)
    sums = lax.dot_general(oh, xv, (((0,), (0,)), ((), ())),
                           preferred_element_type=_f32)[:, 0:4]
    cnt = lax.dot_general(oh, jnp.ones((N, 1), _f32), (((0,), (0,)), ((), ())),
                          preferred_element_type=_f32)
    avg = sums / jnp.maximum(cnt, 1.0)
    h = jnp.dot(avg, wd1_ref[...], preferred_element_type=_f32) + bd1_ref[...]
    h = _gelu_exact(h)
    out_ref[...] = jnp.sum(h * wd2_ref[...], axis=1, keepdims=True) + bd2_ref[0, 0]


def _tc_pool(x, batch_col, wd1, bd1, wd2, bd2):
    return pl.pallas_call(
        _pool_body,
        out_shape=jax.ShapeDtypeStruct((G, 1), _f32),
    )(x, batch_col, wd1, bd1, wd2, bd2)


def _fold_bn(st, w_e1, g, b):
    s = st[0]
    mbar1, mbar2 = s[0] / E, s[1] / E
    s11, s12, s22 = s[2] / E, s[3] / E, s[4] / E
    w0, w1 = w_e1[0], w_e1[1]
    mu = mbar1 * w0 + mbar2 * w1
    ex2 = w0 * w0 * s11 + 2.0 * w0 * w1 * s12 + w1 * w1 * s22
    var = ex2 - mu * mu
    rstd = g * jax.lax.rsqrt(var + 1e-5)
    w1p = w_e1 * rstd[None, :]
    b1p = b - mu * rstd
    return w1p, b1p[None, :]


def kernel(x, edge_index, batch, W_e1, bn_g, bn_b, W_e2, b_e2, W_x1, b_x1,
           W_x2, W_m, b_m, Wd1, bd1, Wd2, bd2):
    src = edge_index[0].astype(jnp.int32)
    dst = edge_index[1].astype(jnp.int32)
    idx_all = jnp.concatenate(
        [src.reshape(NW, NJ, GB), dst.reshape(NW, NJ, GB)],
        axis=1).reshape(NW, 2 * NJ, GB // 16, 16)
    dst_w = dst.reshape(NW, NJ, GB // 16, 16)
    zeros_n = jnp.zeros((N, DP), _f32)
    batch_col = batch.astype(jnp.int32).reshape(N, 1)
    x = jnp.concatenate(
        [x.astype(_f32), jnp.zeros((N, DP - 4), _f32)], axis=1)
    _sc_gather, _sc_scatter = _build_sc_kernels()

    for l in range(L):
        gath = _sc_gather(x, idx_all)
        gv = gath.reshape(NW, 2, EPW, DP)
        m_in, md, st = _tc_stats(gv)
        w1p, b1p = _fold_bn(st, W_e1[l], bn_g[l], bn_b[l])
        msg = _tc_mlp(m_in, md, w1p, b1p, W_e2[l], b_e2[l][None, :],
                      W_m[l], b_m[l].reshape(1, 1),
                      W_x1[l], b_x1[l][None, :], W_x2[l])
        parts = _sc_scatter(x, zeros_n, msg.reshape(NW, NJ, GB, DP), dst_w)
        x = parts[0] + parts[1]

    return _tc_pool(x, batch_col, Wd1, bd1[None, :], Wd2.reshape(1, H),
                    bd2.reshape(1, 1))


# speedup vs baseline: 11.0063x; 3.6083x over previous
import functools

import jax
import jax.numpy as jnp
from jax import lax
from jax.experimental import pallas as pl
from jax.experimental.pallas import tpu as pltpu
from jax.experimental.pallas import tpu_sc as plsc

N = 10000
E = 320000
H = 128
L = 3
G = 128
C = 0.001

NC = 2
NS = 16
NW = NC * NS
EPW = E // NW
GB = 80
NJ = EPW // GB
CH = 2000
DP = 8
VL = 16

_f32 = jnp.float32
_i32 = jnp.int32


def _psi(p):
    return jnp.sign(p) * jnp.log(jnp.abs(p) + 1.0)


def _sc_gather_body(x_hbm, src_hbm, dst_hbm, out_hbm, x_v, is_v, id_v, stage):
    wid = lax.axis_index("s") * NC + lax.axis_index("c")
    e0w = wid * EPW
    pltpu.sync_copy(x_hbm, x_v)
    pltpu.sync_copy(src_hbm.at[pl.ds(e0w, EPW)], is_v)
    pltpu.sync_copy(dst_hbm.at[pl.ds(e0w, EPW)], id_v)
    zero16 = lax.iota(_i32, VL) * 0
    zf = zero16.astype(_f32)

    for ch in range(EPW // CH):
        def group(g, carry):
            e = ch * CH + g * VL
            s16 = is_v[pl.ds(e, VL)]
            d16 = id_v[pl.ds(e, VL)]
            nr = zf
            dr = zf
            col = pl.ds(g * VL, VL)
            for c in range(4):
                c16 = zero16 + c
                xs = plsc.load_gather(x_v, [s16, c16])
                xd = plsc.load_gather(x_v, [d16, c16])
                dd = xs - xd
                sgn = 1.0 if c == 0 else -1.0
                nr = nr + sgn * dd * dd
                dr = dr + sgn * xs * xd
                stage[2 + c, col] = xd - xs
            stage[0, col] = nr
            stage[1, col] = dr
            stage[6, col] = zf
            stage[7, col] = zf
            return carry

        lax.fori_loop(0, CH // VL, group, 0)
        pltpu.sync_copy(stage, out_hbm.at[:, pl.ds(e0w + ch * CH, CH)])


def _sc_scatter_body(x_hbm, z_hbm, msg_hbm, dst_hbm, out_hbm,
                     msg_v, idx_v, idxb, rows, acc):
    cid = lax.axis_index("c")
    sid = lax.axis_index("s")
    wid = sid * NC + cid
    e0w = wid * EPW

    @pl.when(sid == 0)
    def _():
        @pl.when(cid == 0)
        def _():
            pltpu.sync_copy(x_hbm, acc)

        @pl.when(cid != 0)
        def _():
            pltpu.sync_copy(z_hbm, acc)

    plsc.subcore_barrier()
    pltpu.sync_copy(msg_hbm.at[0:4, pl.ds(e0w, EPW)], msg_v)
    pltpu.sync_copy(dst_hbm.at[pl.ds(e0w, EPW)], idx_v)

    iota16 = lax.iota(_i32, VL)
    zero16 = iota16 * 0
    zf = zero16.astype(_f32)
    for k in range(GB // VL):
        for c in range(4, DP):
            plsc.store_scatter(rows, [iota16 + k * VL, zero16 + c], zf)

    def body(j, carry):
        for k in range(GB // VL):
            sl = pl.ds(j * GB + k * VL, VL)
            idxb[pl.ds(k * VL, VL)] = idx_v[sl]
            b16 = iota16 + k * VL
            for c in range(4):
                plsc.store_scatter(rows, [b16, zero16 + c], msg_v[c, sl])
        pltpu.sync_copy(rows, acc.at[idxb], add=True)
        return carry

    lax.fori_loop(0, NJ, body, 0)
    plsc.subcore_barrier()

    @pl.when(sid == 0)
    def _():
        pltpu.sync_copy(acc, out_hbm.at[cid])


@functools.cache
def _build_sc_kernels():
    mesh = plsc.VectorSubcoreMesh(
        core_axis_name="c", subcore_axis_name="s",
        num_cores=NC, num_subcores=NS)
    params = pltpu.CompilerParams(
        use_tc_tiling_on_sc=False, needs_layout_passes=False)
    gather = pl.kernel(
        _sc_gather_body,
        out_type=jax.ShapeDtypeStruct((DP, E), _f32),
        mesh=mesh,
        compiler_params=params,
        scratch_types=(
            pltpu.VMEM((N, DP), _f32),
            pltpu.VMEM((EPW,), _i32),
            pltpu.VMEM((EPW,), _i32),
            pltpu.VMEM((DP, CH), _f32),
        ),
    )
    scatter = pl.kernel(
        _sc_scatter_body,
        out_type=jax.ShapeDtypeStruct((NC, N, DP), _f32),
        mesh=mesh,
        compiler_params=params,
        scratch_types=(
            pltpu.VMEM((4, EPW), _f32),
            pltpu.VMEM((EPW,), _i32),
            pltpu.VMEM((GB,), _i32),
            pltpu.VMEM((GB, DP), _f32),
            pltpu.VMEM_SHARED((N, DP), _f32),
        ),
    )
    return gather, scatter


BSE = 32000


def _stats_body(g_ref, st_ref, acc):
    i = pl.program_id(0)
    m1 = _psi(g_ref[0:1, :])
    m2 = _psi(g_ref[1:2, :])

    @pl.when(i == 0)
    def _():
        acc[...] = jnp.zeros_like(acc)

    acc[:, 0:1] += jnp.sum(m1, axis=1, keepdims=True)
    acc[:, 1:2] += jnp.sum(m2, axis=1, keepdims=True)
    acc[:, 2:3] += jnp.sum(m1 * m1, axis=1, keepdims=True)
    acc[:, 3:4] += jnp.sum(m1 * m2, axis=1, keepdims=True)
    acc[:, 4:5] += jnp.sum(m2 * m2, axis=1, keepdims=True)

    @pl.when(i == pl.num_programs(0) - 1)
    def _():
        st_ref[...] = acc[...]


def _tc_stats(gout):
    return pl.pallas_call(
        _stats_body,
        grid=(E // BSE,),
        in_specs=[pl.BlockSpec((DP, BSE), lambda i: (0, i))],
        out_specs=pl.BlockSpec((1, 8), lambda i: (0, 0)),
        out_shape=jax.ShapeDtypeStruct((1, 8), _f32),
        scratch_shapes=[pltpu.VMEM((1, 8), _f32)],
    )(gout)


BM3 = 16000
_DN = (((0,), (0,)), ((), ()))


def _mlp_body(g_ref, w1_ref, b1_ref, w2_ref, b2_ref, wm_ref, bm_ref,
              wx1_ref, bx1_ref, wx2_ref, out_ref):
    m = jnp.concatenate(
        [_psi(g_ref[0:1, :]), _psi(g_ref[1:2, :])], axis=0)
    h = lax.dot_general(w1_ref[...], m, _DN, preferred_element_type=_f32)
    h = jnp.maximum(h + b1_ref[...], 0.0)
    mij = lax.dot_general(w2_ref[...], h, _DN, preferred_element_type=_f32)
    mij = jnp.maximum(mij + b2_ref[...], 0.0)
    gate = jax.nn.sigmoid(
        lax.dot_general(wm_ref[...], mij, _DN, preferred_element_type=_f32)
        + bm_ref[0, 0])
    mij = mij * gate
    t = lax.dot_general(wx1_ref[...], mij, _DN, preferred_element_type=_f32)
    t = jnp.maximum(t + bx1_ref[...], 0.0)
    scale = lax.dot_general(wx2_ref[...], t, _DN, preferred_element_type=_f32)
    out_ref[0:4, :] = (C * scale) * g_ref[2:6, :]
    out_ref[4:DP, :] = jnp.zeros((DP - 4, out_ref.shape[1]), _f32)


def _tc_mlp(gout, w1p, b1p, w2, b2, wm, bm, wx1, bx1, wx2):
    def full(r, c):
        return pl.BlockSpec((r, c), lambda i: (0, 0))
    return pl.pallas_call(
        _mlp_body,
        grid=(E // BM3,),
        in_specs=[
            pl.BlockSpec((DP, BM3), lambda i: (0, i)),
            full(2, H), full(H, 1), full(H, H), full(H, 1),
            full(H, 1), full(1, 1), full(H, H), full(H, 1), full(H, 1),
        ],
        out_specs=pl.BlockSpec((DP, BM3), lambda i: (0, i)),
        out_shape=jax.ShapeDtypeStruct((DP, E), _f32),
    )(gout, w1p, b1p, w2, b2, wm, bm, wx1, bx1, wx2)


def _erf(x):
    s = jnp.sign(x)
    a = jnp.abs(x)
    t = 1.0 / (1.0 + 0.3275911 * a)
    poly = t * (0.254829592 + t * (-0.284496736 + t * (1.421413741
           + t * (-1.453152027 + t * 1.061405429))))
    return s * (1.0 - poly * jnp.exp(-a * a))


def _gelu_exact(x):
    return 0.5 * x * (1.0 + _erf(x * 0.7071067811865476))


def _pool_body(x_ref, b_ref, wd1_ref, bd1_ref, wd2_ref, bd2_ref, out_ref):
    xv = x_ref[...]
    bv = b_ref[...]
    oh = (bv == lax.broadcasted_iota(jnp.int32, (1, G), 1)).astype(_f32)
    sums = lax.dot_general(oh, xv, (((0,), (0,)), ((), ())),
                           preferred_element_type=_f32)[:, 0:4]
    cnt = lax.dot_general(oh, jnp.ones((N, 1), _f32), (((0,), (0,)), ((), ())),
                          preferred_element_type=_f32)
    avg = sums / jnp.maximum(cnt, 1.0)
    h = jnp.dot(avg, wd1_ref[...], preferred_element_type=_f32) + bd1_ref[...]
    h = _gelu_exact(h)
    out_ref[...] = jnp.sum(h * wd2_ref[...], axis=1, keepdims=True) + bd2_ref[0, 0]


def _tc_pool(x, batch_col, wd1, bd1, wd2, bd2):
    return pl.pallas_call(
        _pool_body,
        out_shape=jax.ShapeDtypeStruct((G, 1), _f32),
    )(x, batch_col, wd1, bd1, wd2, bd2)


def _fold_bn(st, w_e1, g, b):
    s = st[0]
    mbar1, mbar2 = s[0] / E, s[1] / E
    s11, s12, s22 = s[2] / E, s[3] / E, s[4] / E
    w0, w1 = w_e1[0], w_e1[1]
    mu = mbar1 * w0 + mbar2 * w1
    ex2 = w0 * w0 * s11 + 2.0 * w0 * w1 * s12 + w1 * w1 * s22
    var = ex2 - mu * mu
    rstd = g * jax.lax.rsqrt(var + 1e-5)
    w1p = w_e1 * rstd[None, :]
    b1p = b - mu * rstd
    return w1p, b1p[:, None]


def kernel(x, edge_index, batch, W_e1, bn_g, bn_b, W_e2, b_e2, W_x1, b_x1,
           W_x2, W_m, b_m, Wd1, bd1, Wd2, bd2):
    src = edge_index[0].astype(_i32)
    dst = edge_index[1].astype(_i32)
    zeros_n = jnp.zeros((N, DP), _f32)
    batch_col = batch.astype(_i32).reshape(N, 1)
    x8 = jnp.concatenate(
        [x.astype(_f32), jnp.zeros((N, DP - 4), _f32)], axis=1)
    _sc_gather, _sc_scatter = _build_sc_kernels()

    for l in range(L):
        gout = _sc_gather(x8, src, dst)
        st = _tc_stats(gout)
        w1p, b1p = _fold_bn(st, W_e1[l], bn_g[l], bn_b[l])
        msg = _tc_mlp(gout, w1p, b1p, W_e2[l], b_e2[l][:, None],
                      W_m[l], b_m[l].reshape(1, 1),
                      W_x1[l], b_x1[l][:, None], W_x2[l])
        parts = _sc_scatter(x8, zeros_n, msg, dst)
        x8 = parts[0] + parts[1]

    return _tc_pool(x8, batch_col, Wd1, bd1[None, :], Wd2.reshape(1, H),
                    bd2.reshape(1, 1))
